# baseline (speedup 1.0000x reference)
import functools

import jax
import jax.numpy as jnp
from jax import lax
from jax.experimental import pallas as pl
from jax.experimental.pallas import tpu as pltpu
from jax.experimental.pallas import tpu_sc as plsc

EPS = 1e-5

_NC = 2
_NS = 16
_NW = _NC * _NS


def _sc_gather(table, flat_ids, tok0, n_tok):
    d = table.shape[1]
    b_per_w = n_tok // _NW
    chunk = min(64, b_per_w)
    n_chunks = b_per_w // chunk
    mesh = plsc.VectorSubcoreMesh(core_axis_name="c", subcore_axis_name="s")

    @functools.partial(
        pl.kernel,
        mesh=mesh,
        out_type=jax.ShapeDtypeStruct((n_tok, d), jnp.float32),
        scratch_types=[
            pltpu.VMEM((b_per_w,), jnp.int32),
            pltpu.VMEM((chunk, d), jnp.float32),
            pltpu.VMEM((chunk, d), jnp.float32),
            pltpu.SemaphoreType.DMA,
            pltpu.SemaphoreType.DMA,
            pltpu.SemaphoreType.DMA,
            pltpu.SemaphoreType.DMA,
        ],
    )
    def gather_kernel(table_hbm, idx_hbm, out_hbm, idx_v, rows0, rows1,
                      g0, g1, w0, w1):
        wid = lax.axis_index("s") * _NC + lax.axis_index("c")
        base = wid * b_per_w
        bufs, gsems, wsems = [rows0, rows1], [g0, g1], [w0, w1]
        pltpu.sync_copy(idx_hbm.at[pl.ds(tok0 + base, b_per_w)], idx_v)

        def gather_start(c):
            idx_c = idx_v.at[pl.ds(c * chunk, chunk)]
            return pltpu.async_copy(table_hbm.at[idx_c], bufs[c % 2],
                                    gsems[c % 2])

        def write_start(c):
            return pltpu.async_copy(bufs[c % 2],
                                    out_hbm.at[pl.ds(base + c * chunk, chunk)],
                                    wsems[c % 2])

        gathers = [gather_start(0)]
        writes = [None, None]
        for c in range(n_chunks):
            gathers[c].wait()
            if c >= 1:
                writes[(c - 1) % 2].wait()
            if c + 1 < n_chunks:
                gathers.append(gather_start(c + 1))
            writes[c % 2] = write_start(c)
        writes[(n_chunks - 1) % 2].wait()

    return gather_kernel(table, flat_ids)


def _ln_compute(w_ref, t_ref, pos_ref, ttab_ref, sc_ref, of_ref, out_ref):
    bb = w_ref.shape[0]
    base = pos_ref[...] + ttab_ref[0:1, :]
    diff = ttab_ref[1:2, :] - ttab_ref[0:1, :]
    for j in range(bb):
        tf = t_ref[j].T.astype(jnp.float32)
        x = w_ref[j] + base + tf * diff
        mean = jnp.mean(x, axis=1, keepdims=True)
        xc = x - mean
        var = jnp.mean(xc * xc, axis=1, keepdims=True)
        y = xc * lax.rsqrt(var + EPS)
        out_ref[j] = y * sc_ref[...] + of_ref[...]


def _ln_compute_aliased(buf_ref, w_ref, t_ref, pos_ref, ttab_ref, sc_ref,
                        of_ref, out_ref):
    del buf_ref
    _ln_compute(w_ref, t_ref, pos_ref, ttab_ref, sc_ref, of_ref, out_ref)


def _tc_add_ln(word_emb, tt3, pos, ttab, sc2, of2, b_total, c0, out_buf):
    bc, s, d = word_emb.shape
    bb = 4
    off = c0 // bb
    coff = c0 // bb
    in_specs = [
        pl.BlockSpec((bb, s, d), lambda i: (i, 0, 0)),
        pl.BlockSpec((bb, 1, s), lambda i, _o=coff: (i + _o, 0, 0)),
        pl.BlockSpec((s, d), lambda i: (0, 0)),
        pl.BlockSpec((2, d), lambda i: (0, 0)),
        pl.BlockSpec((1, d), lambda i: (0, 0)),
        pl.BlockSpec((1, d), lambda i: (0, 0)),
    ]
    args = (word_emb, tt3, pos, ttab, sc2, of2)
    out_spec = pl.BlockSpec((bb, s, d), lambda i, _o=off: (i + _o, 0, 0))
    out_shape = jax.ShapeDtypeStruct((b_total, s, d), jnp.float32)
    if out_buf is None:
        return pl.pallas_call(
            _ln_compute, grid=(bc // bb,), in_specs=in_specs,
            out_specs=out_spec, out_shape=out_shape)(*args)
    return pl.pallas_call(
        _ln_compute_aliased, grid=(bc // bb,),
        in_specs=[pl.BlockSpec(memory_space=pl.ANY)] + in_specs,
        out_specs=out_spec, out_shape=out_shape,
        input_output_aliases={0: 0})(out_buf, *args)


@jax.jit
def kernel(input_ids, token_type_ids, word_table, pos_table, type_table, ln_scale, ln_offset):
    b, s = input_ids.shape
    d = word_table.shape[1]
    plan = [8, 12, 12]
    flat_ids = input_ids.reshape(b * s)
    tt3 = token_type_ids.reshape(b, 1, s)
    pos = pos_table[:s]
    sc2 = ln_scale.reshape(1, d)
    of2 = ln_offset.reshape(1, d)
    out = None
    c0 = 0
    for bc in plan:
        w_c = _sc_gather(word_table, flat_ids, c0 * s, bc * s).reshape(bc, s, d)
        out = _tc_add_ln(w_c, tt3, pos, type_table, sc2, of2, b, c0, out)
        c0 += bc
    return out


# speedup vs baseline: 1.0114x; 1.0114x over previous
import functools

import jax
import jax.numpy as jnp
from jax import lax
from jax.experimental import pallas as pl
from jax.experimental.pallas import tpu as pltpu
from jax.experimental.pallas import tpu_sc as plsc

EPS = 1e-5

_NC = 2
_NS = 16
_NW = _NC * _NS


def _sc_gather(table, flat_ids, tok0, n_tok):
    d = table.shape[1]
    b_per_w = n_tok // _NW
    chunk = min(64, b_per_w)
    n_chunks = b_per_w // chunk
    mesh = plsc.VectorSubcoreMesh(core_axis_name="c", subcore_axis_name="s")

    @functools.partial(
        pl.kernel,
        mesh=mesh,
        out_type=jax.ShapeDtypeStruct((n_tok, d), jnp.float32),
        scratch_types=[
            pltpu.VMEM((b_per_w,), jnp.int32),
            pltpu.VMEM((chunk, d), jnp.float32),
            pltpu.VMEM((chunk, d), jnp.float32),
            pltpu.SemaphoreType.DMA,
            pltpu.SemaphoreType.DMA,
            pltpu.SemaphoreType.DMA,
            pltpu.SemaphoreType.DMA,
        ],
    )
    def gather_kernel(table_hbm, idx_hbm, out_hbm, idx_v, rows0, rows1,
                      g0, g1, w0, w1):
        wid = lax.axis_index("s") * _NC + lax.axis_index("c")
        base = wid * b_per_w
        bufs, gsems, wsems = [rows0, rows1], [g0, g1], [w0, w1]
        pltpu.sync_copy(idx_hbm.at[pl.ds(tok0 + base, b_per_w)], idx_v)

        def gather_start(c):
            idx_c = idx_v.at[pl.ds(c * chunk, chunk)]
            return pltpu.async_copy(table_hbm.at[idx_c], bufs[c % 2],
                                    gsems[c % 2])

        def write_start(c):
            return pltpu.async_copy(bufs[c % 2],
                                    out_hbm.at[pl.ds(base + c * chunk, chunk)],
                                    wsems[c % 2])

        gathers = [gather_start(0)]
        writes = [None, None]
        for c in range(n_chunks):
            gathers[c].wait()
            if c >= 1:
                writes[(c - 1) % 2].wait()
            if c + 1 < n_chunks:
                gathers.append(gather_start(c + 1))
            writes[c % 2] = write_start(c)
        writes[(n_chunks - 1) % 2].wait()

    return gather_kernel(table, flat_ids)


def _ln_compute(w_ref, t_ref, pos_ref, ttab_ref, sc_ref, of_ref, out_ref):
    bb = w_ref.shape[0]
    base = pos_ref[...] + ttab_ref[0:1, :]
    diff = ttab_ref[1:2, :] - ttab_ref[0:1, :]
    for j in range(bb):
        tf = t_ref[j].T.astype(jnp.float32)
        x = w_ref[j] + base + tf * diff
        mean = jnp.mean(x, axis=1, keepdims=True)
        xc = x - mean
        var = jnp.mean(xc * xc, axis=1, keepdims=True)
        y = xc * lax.rsqrt(var + EPS)
        out_ref[j] = y * sc_ref[...] + of_ref[...]


def _ln_compute_aliased(buf_ref, w_ref, t_ref, pos_ref, ttab_ref, sc_ref,
                        of_ref, out_ref):
    del buf_ref
    _ln_compute(w_ref, t_ref, pos_ref, ttab_ref, sc_ref, of_ref, out_ref)


def _tc_add_ln(word_emb, tt3, pos, ttab, sc2, of2, b_total, c0, out_buf):
    bc, s, d = word_emb.shape
    bb = 4
    off = c0 // bb
    coff = c0 // bb
    in_specs = [
        pl.BlockSpec((bb, s, d), lambda i: (i, 0, 0)),
        pl.BlockSpec((bb, 1, s), lambda i, _o=coff: (i + _o, 0, 0)),
        pl.BlockSpec((s, d), lambda i: (0, 0)),
        pl.BlockSpec((2, d), lambda i: (0, 0)),
        pl.BlockSpec((1, d), lambda i: (0, 0)),
        pl.BlockSpec((1, d), lambda i: (0, 0)),
    ]
    args = (word_emb, tt3, pos, ttab, sc2, of2)
    out_spec = pl.BlockSpec((bb, s, d), lambda i, _o=off: (i + _o, 0, 0))
    out_shape = jax.ShapeDtypeStruct((b_total, s, d), jnp.float32)
    if out_buf is None:
        return pl.pallas_call(
            _ln_compute, grid=(bc // bb,), in_specs=in_specs,
            out_specs=out_spec, out_shape=out_shape)(*args)
    return pl.pallas_call(
        _ln_compute_aliased, grid=(bc // bb,),
        in_specs=[pl.BlockSpec(memory_space=pl.ANY)] + in_specs,
        out_specs=out_spec, out_shape=out_shape,
        input_output_aliases={0: 0})(out_buf, *args)


@jax.jit
def kernel(input_ids, token_type_ids, word_table, pos_table, type_table, ln_scale, ln_offset):
    b, s = input_ids.shape
    d = word_table.shape[1]
    plan = [12, 20]
    flat_ids = input_ids.reshape(b * s)
    tt3 = token_type_ids.reshape(b, 1, s)
    pos = pos_table[:s]
    sc2 = ln_scale.reshape(1, d)
    of2 = ln_offset.reshape(1, d)
    out = None
    c0 = 0
    for bc in plan:
        w_c = _sc_gather(word_table, flat_ids, c0 * s, bc * s).reshape(bc, s, d)
        out = _tc_add_ln(w_c, tt3, pos, type_table, sc2, of2, b, c0, out)
        c0 += bc
    return out


# speedup vs baseline: 1.0174x; 1.0060x over previous
import functools

import jax
import jax.numpy as jnp
from jax import lax
from jax.experimental import pallas as pl
from jax.experimental.pallas import tpu as pltpu
from jax.experimental.pallas import tpu_sc as plsc

EPS = 1e-5

_NC = 2
_NS = 16
_NW = _NC * _NS


def _sc_gather(table, flat_ids, tok0, n_tok):
    d = table.shape[1]
    b_per_w = n_tok // _NW
    chunk = min(64, b_per_w)
    n_chunks = b_per_w // chunk
    mesh = plsc.VectorSubcoreMesh(core_axis_name="c", subcore_axis_name="s")

    @functools.partial(
        pl.kernel,
        mesh=mesh,
        out_type=jax.ShapeDtypeStruct((n_tok, d), jnp.float32),
        scratch_types=[
            pltpu.VMEM((b_per_w,), jnp.int32),
            pltpu.VMEM((chunk, d), jnp.float32),
            pltpu.VMEM((chunk, d), jnp.float32),
            pltpu.SemaphoreType.DMA,
            pltpu.SemaphoreType.DMA,
            pltpu.SemaphoreType.DMA,
            pltpu.SemaphoreType.DMA,
        ],
    )
    def gather_kernel(table_hbm, idx_hbm, out_hbm, idx_v, rows0, rows1,
                      g0, g1, w0, w1):
        wid = lax.axis_index("s") * _NC + lax.axis_index("c")
        base = wid * b_per_w
        bufs, gsems, wsems = [rows0, rows1], [g0, g1], [w0, w1]
        pltpu.sync_copy(idx_hbm.at[pl.ds(tok0 + base, b_per_w)], idx_v)

        def gather_start(c):
            idx_c = idx_v.at[pl.ds(c * chunk, chunk)]
            return pltpu.async_copy(table_hbm.at[idx_c], bufs[c % 2],
                                    gsems[c % 2])

        def write_start(c):
            return pltpu.async_copy(bufs[c % 2],
                                    out_hbm.at[pl.ds(base + c * chunk, chunk)],
                                    wsems[c % 2])

        gathers = [gather_start(0)]
        writes = [None, None]
        for c in range(n_chunks):
            gathers[c].wait()
            if c >= 1:
                writes[(c - 1) % 2].wait()
            if c + 1 < n_chunks:
                gathers.append(gather_start(c + 1))
            writes[c % 2] = write_start(c)
        writes[(n_chunks - 1) % 2].wait()

    return gather_kernel(table, flat_ids)


def _ln_compute(w_ref, t_ref, pos_ref, ttab_ref, sc_ref, of_ref, out_ref):
    bb = w_ref.shape[0]
    base = pos_ref[...] + ttab_ref[0:1, :]
    diff = ttab_ref[1:2, :] - ttab_ref[0:1, :]
    for j in range(bb):
        tf = t_ref[j].T.astype(jnp.float32)
        x = w_ref[j] + base + tf * diff
        mean = jnp.mean(x, axis=1, keepdims=True)
        xc = x - mean
        var = jnp.mean(xc * xc, axis=1, keepdims=True)
        y = xc * lax.rsqrt(var + EPS)
        out_ref[j] = y * sc_ref[...] + of_ref[...]


def _ln_compute_aliased(buf_ref, w_ref, t_ref, pos_ref, ttab_ref, sc_ref,
                        of_ref, out_ref):
    del buf_ref
    _ln_compute(w_ref, t_ref, pos_ref, ttab_ref, sc_ref, of_ref, out_ref)


def _tc_add_ln(word_emb, tt3, pos, ttab, sc2, of2, b_total, c0, out_buf):
    bc, s, d = word_emb.shape
    bb = 4
    off = c0 // bb
    coff = c0 // bb
    in_specs = [
        pl.BlockSpec((bb, s, d), lambda i: (i, 0, 0)),
        pl.BlockSpec((bb, 1, s), lambda i, _o=coff: (i + _o, 0, 0)),
        pl.BlockSpec((s, d), lambda i: (0, 0)),
        pl.BlockSpec((2, d), lambda i: (0, 0)),
        pl.BlockSpec((1, d), lambda i: (0, 0)),
        pl.BlockSpec((1, d), lambda i: (0, 0)),
    ]
    args = (word_emb, tt3, pos, ttab, sc2, of2)
    out_spec = pl.BlockSpec((bb, s, d), lambda i, _o=off: (i + _o, 0, 0))
    out_shape = jax.ShapeDtypeStruct((b_total, s, d), jnp.float32)
    if out_buf is None:
        return pl.pallas_call(
            _ln_compute, grid=(bc // bb,), in_specs=in_specs,
            out_specs=out_spec, out_shape=out_shape)(*args)
    return pl.pallas_call(
        _ln_compute_aliased, grid=(bc // bb,),
        in_specs=[pl.BlockSpec(memory_space=pl.ANY)] + in_specs,
        out_specs=out_spec, out_shape=out_shape,
        input_output_aliases={0: 0})(out_buf, *args)


@jax.jit
def kernel(input_ids, token_type_ids, word_table, pos_table, type_table, ln_scale, ln_offset):
    b, s = input_ids.shape
    d = word_table.shape[1]
    plan = [20, 12]
    flat_ids = input_ids.reshape(b * s)
    tt3 = token_type_ids.reshape(b, 1, s)
    pos = pos_table[:s]
    sc2 = ln_scale.reshape(1, d)
    of2 = ln_offset.reshape(1, d)
    out = None
    c0 = 0
    for bc in plan:
        w_c = _sc_gather(word_table, flat_ids, c0 * s, bc * s).reshape(bc, s, d)
        out = _tc_add_ln(w_c, tt3, pos, type_table, sc2, of2, b, c0, out)
        c0 += bc
    return out


# speedup vs baseline: 1.0274x; 1.0098x over previous
import functools

import jax
import jax.numpy as jnp
from jax import lax
from jax.experimental import pallas as pl
from jax.experimental.pallas import tpu as pltpu
from jax.experimental.pallas import tpu_sc as plsc

EPS = 1e-5

_NC = 2
_NS = 16
_NW = _NC * _NS


def _sc_gather(table, flat_ids, tok0, n_tok):
    d = table.shape[1]
    b_per_w = n_tok // _NW
    chunk = min(64, b_per_w)
    n_chunks = b_per_w // chunk
    mesh = plsc.VectorSubcoreMesh(core_axis_name="c", subcore_axis_name="s")

    @functools.partial(
        pl.kernel,
        mesh=mesh,
        out_type=jax.ShapeDtypeStruct((n_tok, d), jnp.float32),
        scratch_types=[
            pltpu.VMEM((b_per_w,), jnp.int32),
            pltpu.VMEM((chunk, d), jnp.float32),
            pltpu.VMEM((chunk, d), jnp.float32),
            pltpu.SemaphoreType.DMA,
            pltpu.SemaphoreType.DMA,
            pltpu.SemaphoreType.DMA,
            pltpu.SemaphoreType.DMA,
        ],
    )
    def gather_kernel(table_hbm, idx_hbm, out_hbm, idx_v, rows0, rows1,
                      g0, g1, w0, w1):
        wid = lax.axis_index("s") * _NC + lax.axis_index("c")
        base = wid * b_per_w
        bufs, gsems, wsems = [rows0, rows1], [g0, g1], [w0, w1]
        pltpu.sync_copy(idx_hbm.at[pl.ds(tok0 + base, b_per_w)], idx_v)

        def gather_start(c):
            idx_c = idx_v.at[pl.ds(c * chunk, chunk)]
            return pltpu.async_copy(table_hbm.at[idx_c], bufs[c % 2],
                                    gsems[c % 2])

        def write_start(c):
            return pltpu.async_copy(bufs[c % 2],
                                    out_hbm.at[pl.ds(base + c * chunk, chunk)],
                                    wsems[c % 2])

        gathers = [gather_start(0)]
        writes = [None, None]
        for c in range(n_chunks):
            gathers[c].wait()
            if c >= 1:
                writes[(c - 1) % 2].wait()
            if c + 1 < n_chunks:
                gathers.append(gather_start(c + 1))
            writes[c % 2] = write_start(c)
        writes[(n_chunks - 1) % 2].wait()

    return gather_kernel(table, flat_ids)


def _ln_compute(w_ref, t_ref, pos_ref, ttab_ref, sc_ref, of_ref, out_ref):
    bb = w_ref.shape[0]
    base = pos_ref[...] + ttab_ref[0:1, :]
    diff = ttab_ref[1:2, :] - ttab_ref[0:1, :]
    for j in range(bb):
        tf = t_ref[j].T.astype(jnp.float32)
        x = w_ref[j] + base + tf * diff
        mean = jnp.mean(x, axis=1, keepdims=True)
        xc = x - mean
        var = jnp.mean(xc * xc, axis=1, keepdims=True)
        y = xc * lax.rsqrt(var + EPS)
        out_ref[j] = y * sc_ref[...] + of_ref[...]


def _ln_compute_aliased(buf_ref, w_ref, t_ref, pos_ref, ttab_ref, sc_ref,
                        of_ref, out_ref):
    del buf_ref
    _ln_compute(w_ref, t_ref, pos_ref, ttab_ref, sc_ref, of_ref, out_ref)


def _tc_add_ln(word_emb, tt3, pos, ttab, sc2, of2, b_total, c0, out_buf):
    bc, s, d = word_emb.shape
    bb = 4
    off = c0 // bb
    coff = c0 // bb
    in_specs = [
        pl.BlockSpec((bb, s, d), lambda i: (i, 0, 0)),
        pl.BlockSpec((bb, 1, s), lambda i, _o=coff: (i + _o, 0, 0)),
        pl.BlockSpec((s, d), lambda i: (0, 0)),
        pl.BlockSpec((2, d), lambda i: (0, 0)),
        pl.BlockSpec((1, d), lambda i: (0, 0)),
        pl.BlockSpec((1, d), lambda i: (0, 0)),
    ]
    args = (word_emb, tt3, pos, ttab, sc2, of2)
    out_spec = pl.BlockSpec((bb, s, d), lambda i, _o=off: (i + _o, 0, 0))
    out_shape = jax.ShapeDtypeStruct((b_total, s, d), jnp.float32)
    if out_buf is None:
        return pl.pallas_call(
            _ln_compute, grid=(bc // bb,), in_specs=in_specs,
            out_specs=out_spec, out_shape=out_shape)(*args)
    return pl.pallas_call(
        _ln_compute_aliased, grid=(bc // bb,),
        in_specs=[pl.BlockSpec(memory_space=pl.ANY)] + in_specs,
        out_specs=out_spec, out_shape=out_shape,
        input_output_aliases={0: 0})(out_buf, *args)


@jax.jit
def kernel(input_ids, token_type_ids, word_table, pos_table, type_table, ln_scale, ln_offset):
    b, s = input_ids.shape
    d = word_table.shape[1]
    plan = [16, 16]
    flat_ids = input_ids.reshape(b * s)
    tt3 = token_type_ids.reshape(b, 1, s)
    pos = pos_table[:s]
    sc2 = ln_scale.reshape(1, d)
    of2 = ln_offset.reshape(1, d)
    out = None
    c0 = 0
    for bc in plan:
        w_c = _sc_gather(word_table, flat_ids, c0 * s, bc * s).reshape(bc, s, d)
        out = _tc_add_ln(w_c, tt3, pos, type_table, sc2, of2, b, c0, out)
        c0 += bc
    return out
